# Initial kernel scaffold; baseline (speedup 1.0000x reference)
import jax
import jax.numpy as jnp
from jax.experimental import pallas as pl


def kernel(x, W_qkv, b_qkv, W_o, b_o, W_g, b_g):
    raise NotImplementedError("write your pallas kernel here")


# speedup vs baseline: 4.1055x; 4.1055x over previous
import jax
import jax.numpy as jnp
from jax import lax
from jax.experimental import pallas as pl
from jax.experimental.pallas import tpu as pltpu

DIM = 384
QK_DIM = 384
NUM_HEADS = 8
N_WIN = 8
KV_PER_WIN = 4
ATT_TOPK = 4
SCALE = QK_DIM ** (-0.5)

N = 4
H = 224
W = 224
HW = N_WIN
ROWS = H // N_WIN
CHUNK = 14
NCHUNK = ROWS // CHUNK
C_KV = QK_DIM + DIM

F32 = jnp.float32


def _pool_body(x_ref, out_ref):
    xb = x_ref[0]
    t = xb.reshape(4, 7, W, DIM).sum(axis=1)
    t = t.reshape(4, 4, 56, DIM).sum(axis=2)
    out_ref[0] = (t * (1.0 / (7.0 * 56.0))).reshape(16, DIM)


def _route_body(xp_ref, wqkv_ref, bqkv_ref, mm_ref):
    xp = xp_ref[...]
    wq = wqkv_ref[:, :QK_DIM]
    wkv = wqkv_ref[:, QK_DIM:]
    bq = bqkv_ref[0, :QK_DIM]
    bkv = bqkv_ref[0, QK_DIM:]

    xm = jnp.mean(xp, axis=1)
    qwin = jnp.dot(xm, wq, preferred_element_type=F32) + bq

    logits = []
    for n in range(N):
        qn = qwin[n * HW:(n + 1) * HW, :]
        ln = lax.dot_general(qn, qn, (((1,), (1,)), ((), ())),
                             preferred_element_type=F32) * SCALE
        logits.append(ln)
    lg = jnp.stack(logits, axis=0)

    iota = lax.broadcasted_iota(jnp.int32, (N, HW, HW), 2)
    sel = jnp.zeros((N, HW, HW), F32)
    for _ in range(ATT_TOPK):
        mx = jnp.max(lg, axis=2, keepdims=True)
        ismax = lg >= mx
        fidx = jnp.min(jnp.where(ismax, iota, 127), axis=2, keepdims=True)
        onehot = iota == fidx
        sel = sel + onehot.astype(F32)
        lg = jnp.where(onehot, jnp.float32(-3e38), lg)

    kvp = jnp.dot(xp.reshape(32 * 16, DIM), wkv,
                  preferred_element_type=F32) + bkv
    kk = kvp[:, :QK_DIM].reshape(N, HW, 16, QK_DIM)
    vv = kvp[:, QK_DIM:].reshape(N, HW, 16, DIM)

    ri = lax.broadcasted_iota(jnp.int32, (QK_DIM, DIM), 0) // 48
    ci = lax.broadcasted_iota(jnp.int32, (QK_DIM, DIM), 1) // 48
    mask = (ri == ci).astype(F32)

    for idx in range(N * HW):
        n, w = idx // HW, idx % HW
        srow = sel[n, w]
        ksel = (kk[n] * srow[:, None, None]).reshape(HW * 16, QK_DIM)
        vn = vv[n].reshape(HW * 16, DIM)
        mfull = lax.dot_general(ksel, vn, (((0,), (0,)), ((), ())),
                                preferred_element_type=F32)
        mm_ref[idx] = mfull * mask


def _bmat_body(mm_ref, wqkv_ref, bqkv_ref, wo_ref, bo_ref, bg_ref,
               b_ref, cv_ref):
    a = jnp.dot(mm_ref[0], wo_ref[...], preferred_element_type=F32) * SCALE
    wq = wqkv_ref[:, :QK_DIM]
    b_ref[0] = jnp.dot(wq, a, preferred_element_type=F32)
    bq = bqkv_ref[0:1, :QK_DIM]
    cv = (jnp.dot(bq, a, preferred_element_type=F32)
          + jnp.dot(bg_ref[...], wo_ref[...], preferred_element_type=F32)
          + bo_ref[...])
    cv_ref[0] = cv


def _main_body(b_ref, cv_ref, wv_ref, bv_ref, wg_ref, wo_ref, x_hbm,
               out_ref, xb, sem0, sem1, sem2):
    p = pl.program_id(0)
    c = pl.program_id(1)
    n = p // HW
    row0 = (p % HW) * ROWS + c * CHUNK

    pltpu.make_async_copy(x_hbm.at[n, pl.ds(row0, CHUNK)],
                          xb.at[pl.ds(1, CHUNK)], sem0).start()

    @pl.when(row0 > 0)
    def _():
        pltpu.make_async_copy(x_hbm.at[n, pl.ds(row0 - 1, 1)],
                              xb.at[pl.ds(0, 1)], sem1).start()
        pltpu.make_async_copy(x_hbm.at[n, pl.ds(row0 - 1, 1)],
                              xb.at[pl.ds(0, 1)], sem1).wait()

    @pl.when(row0 == 0)
    def _():
        xb[0] = jnp.zeros((W, DIM), F32)

    @pl.when(row0 + CHUNK < H)
    def _():
        pltpu.make_async_copy(x_hbm.at[n, pl.ds(row0 + CHUNK, 1)],
                              xb.at[pl.ds(CHUNK + 1, 1)], sem2).start()
        pltpu.make_async_copy(x_hbm.at[n, pl.ds(row0 + CHUNK, 1)],
                              xb.at[pl.ds(CHUNK + 1, 1)], sem2).wait()

    @pl.when(row0 + CHUNK == H)
    def _():
        xb[CHUNK + 1] = jnp.zeros((W, DIM), F32)

    pltpu.make_async_copy(x_hbm.at[n, pl.ds(row0, CHUNK)],
                          xb.at[pl.ds(1, CHUNK)], sem0).wait()

    xall = xb[...]
    v = jnp.dot(xall.reshape((CHUNK + 2) * W, DIM), wv_ref[...],
                preferred_element_type=F32) + bv_ref[0]
    v3 = v.reshape(CHUNK + 2, W, DIM)
    zcol = jnp.zeros((CHUNK + 2, 1, DIM), F32)
    vp = jnp.concatenate([zcol, v3, zcol], axis=1)

    g = None
    for ky in range(3):
        for kx in range(3):
            term = vp[ky:ky + CHUNK, kx:kx + W, :] * wg_ref[ky, kx]
            g = term if g is None else g + term

    xc = xall[1:CHUNK + 1].reshape(CHUNK * W, DIM)
    out = (jnp.dot(xc, b_ref[0], preferred_element_type=F32)
           + jnp.dot(g.reshape(CHUNK * W, DIM), wo_ref[...],
                     preferred_element_type=F32)
           + cv_ref[0, 0])
    out_ref[0] = out.reshape(CHUNK, W, DIM)


def kernel(x, W_qkv, b_qkv, W_o, b_o, W_g, b_g):
    x32 = x.reshape(N, HW, ROWS, W, DIM).reshape(N * HW, ROWS, W, DIM)

    x_pool = pl.pallas_call(
        _pool_body,
        grid=(N * HW,),
        in_specs=[pl.BlockSpec((1, ROWS, W, DIM), lambda i: (i, 0, 0, 0))],
        out_specs=pl.BlockSpec((1, 16, DIM), lambda i: (i, 0, 0)),
        out_shape=jax.ShapeDtypeStruct((N * HW, 16, DIM), F32),
    )(x32)

    bqkv2 = b_qkv.reshape(1, 2 * QK_DIM + DIM)

    mm = pl.pallas_call(
        _route_body,
        in_specs=[
            pl.BlockSpec((N * HW, 16, DIM), lambda: (0, 0, 0)),
            pl.BlockSpec((DIM, 2 * QK_DIM + DIM), lambda: (0, 0)),
            pl.BlockSpec((1, 2 * QK_DIM + DIM), lambda: (0, 0)),
        ],
        out_specs=pl.BlockSpec((N * HW, QK_DIM, DIM), lambda: (0, 0, 0)),
        out_shape=jax.ShapeDtypeStruct((N * HW, QK_DIM, DIM), F32),
    )(x_pool, W_qkv, bqkv2)

    bo2 = b_o.reshape(1, DIM)
    bg2 = b_g.reshape(1, DIM)

    bmat, cvec = pl.pallas_call(
        _bmat_body,
        grid=(N * HW,),
        in_specs=[
            pl.BlockSpec((1, QK_DIM, DIM), lambda i: (i, 0, 0)),
            pl.BlockSpec((DIM, 2 * QK_DIM + DIM), lambda i: (0, 0)),
            pl.BlockSpec((1, 2 * QK_DIM + DIM), lambda i: (0, 0)),
            pl.BlockSpec((DIM, DIM), lambda i: (0, 0)),
            pl.BlockSpec((1, DIM), lambda i: (0, 0)),
            pl.BlockSpec((1, DIM), lambda i: (0, 0)),
        ],
        out_specs=[
            pl.BlockSpec((1, DIM, DIM), lambda i: (i, 0, 0)),
            pl.BlockSpec((1, 1, DIM), lambda i: (i, 0, 0)),
        ],
        out_shape=[
            jax.ShapeDtypeStruct((N * HW, DIM, DIM), F32),
            jax.ShapeDtypeStruct((N * HW, 1, DIM), F32),
        ],
    )(mm, W_qkv, bqkv2, W_o, bo2, bg2)

    wv = W_qkv[:, QK_DIM + QK_DIM:]
    bv = b_qkv[QK_DIM + QK_DIM:].reshape(1, DIM)
    wg = jnp.transpose(W_g[:, 0, :, :], (1, 2, 0))

    out = pl.pallas_call(
        _main_body,
        grid=(N * HW, NCHUNK),
        in_specs=[
            pl.BlockSpec((1, DIM, DIM), lambda p, c: (p, 0, 0)),
            pl.BlockSpec((1, 1, DIM), lambda p, c: (p, 0, 0)),
            pl.BlockSpec((DIM, DIM), lambda p, c: (0, 0)),
            pl.BlockSpec((1, DIM), lambda p, c: (0, 0)),
            pl.BlockSpec((3, 3, DIM), lambda p, c: (0, 0, 0)),
            pl.BlockSpec((DIM, DIM), lambda p, c: (0, 0)),
            pl.BlockSpec(memory_space=pl.ANY),
        ],
        out_specs=pl.BlockSpec((1, CHUNK, W, DIM),
                               lambda p, c: (p // HW, (p % HW) * NCHUNK + c,
                                             0, 0)),
        out_shape=jax.ShapeDtypeStruct((N, H, W, DIM), F32),
        scratch_shapes=[
            pltpu.VMEM((CHUNK + 2, W, DIM), F32),
            pltpu.SemaphoreType.DMA,
            pltpu.SemaphoreType.DMA,
            pltpu.SemaphoreType.DMA,
        ],
        compiler_params=pltpu.CompilerParams(
            dimension_semantics=("arbitrary", "arbitrary")),
    )(bmat, cvec, wv, bv, wg, W_o, x)

    return out


# speedup vs baseline: 5.5351x; 1.3482x over previous
import jax
import jax.numpy as jnp
from jax import lax
from jax.experimental import pallas as pl
from jax.experimental.pallas import tpu as pltpu

DIM = 384
QK_DIM = 384
NUM_HEADS = 8
N_WIN = 8
KV_PER_WIN = 4
ATT_TOPK = 4
SCALE = QK_DIM ** (-0.5)

N = 4
H = 224
W = 224
HW = N_WIN
ROWS = H // N_WIN
CHUNK = 14
NCHUNK = ROWS // CHUNK
C_KV = QK_DIM + DIM

F32 = jnp.float32


def _pool_body(x_ref, out_ref):
    xb = x_ref[0]
    t = xb.reshape(4, 7, W, DIM).sum(axis=1)
    t = t.reshape(4, 4, 56, DIM).sum(axis=2)
    out_ref[0] = (t * (1.0 / (7.0 * 56.0))).reshape(16, DIM)


def _route_body(xp_ref, wqkv_ref, bqkv_ref, mm_ref):
    xp = xp_ref[...]
    wq = wqkv_ref[:, :QK_DIM]
    wkv = wqkv_ref[:, QK_DIM:]
    bq = bqkv_ref[0, :QK_DIM]
    bkv = bqkv_ref[0, QK_DIM:]

    xm = jnp.mean(xp, axis=1)
    qwin = jnp.dot(xm, wq, preferred_element_type=F32) + bq

    logits = []
    for n in range(N):
        qn = qwin[n * HW:(n + 1) * HW, :]
        ln = lax.dot_general(qn, qn, (((1,), (1,)), ((), ())),
                             preferred_element_type=F32) * SCALE
        logits.append(ln)
    lg = jnp.stack(logits, axis=0)

    iota = lax.broadcasted_iota(jnp.int32, (N, HW, HW), 2)
    sel = jnp.zeros((N, HW, HW), F32)
    for _ in range(ATT_TOPK):
        mx = jnp.max(lg, axis=2, keepdims=True)
        ismax = lg >= mx
        fidx = jnp.min(jnp.where(ismax, iota, 127), axis=2, keepdims=True)
        onehot = iota == fidx
        sel = sel + onehot.astype(F32)
        lg = jnp.where(onehot, jnp.float32(-3e38), lg)

    kvp = jnp.dot(xp.reshape(32 * 16, DIM), wkv,
                  preferred_element_type=F32) + bkv
    kk = kvp[:, :QK_DIM].reshape(N, HW, 16, QK_DIM)
    vv = kvp[:, QK_DIM:].reshape(N, HW, 16, DIM)

    ri = lax.broadcasted_iota(jnp.int32, (QK_DIM, DIM), 0) // 48
    ci = lax.broadcasted_iota(jnp.int32, (QK_DIM, DIM), 1) // 48
    mask = (ri == ci).astype(F32)

    for idx in range(N * HW):
        n, w = idx // HW, idx % HW
        srow = sel[n, w]
        ksel = (kk[n] * srow[:, None, None]).reshape(HW * 16, QK_DIM)
        vn = vv[n].reshape(HW * 16, DIM)
        mfull = lax.dot_general(ksel, vn, (((0,), (0,)), ((), ())),
                                preferred_element_type=F32)
        mm_ref[idx] = mfull * mask


def _bmat_body(mm_ref, wqkv_ref, bqkv_ref, wo_ref, bo_ref, bg_ref,
               b_ref, cv_ref):
    a = jnp.dot(mm_ref[0], wo_ref[...], preferred_element_type=F32) * SCALE
    wq = wqkv_ref[:, :QK_DIM]
    b_ref[0] = jnp.dot(wq, a, preferred_element_type=F32)
    bq = bqkv_ref[0:1, :QK_DIM]
    cv = (jnp.dot(bq, a, preferred_element_type=F32)
          + jnp.dot(bg_ref[...], wo_ref[...], preferred_element_type=F32)
          + bo_ref[...])
    cv_ref[0] = cv


NPROG = N * HW * NCHUNK
BF16 = jnp.bfloat16


def _main_body(b_ref, cv_ref, wv_ref, bv_ref, wg_ref, wo_ref, x_hbm,
               out_ref, xb, semc, semt, semb):
    p = pl.program_id(0)
    c = pl.program_id(1)
    i = p * NCHUNK + c

    def _copies(idx, slot):
        n = idx // (HW * NCHUNK)
        r0 = (idx % (HW * NCHUNK)) * CHUNK
        core = pltpu.make_async_copy(
            x_hbm.at[n, pl.ds(r0, CHUNK)],
            xb.at[slot, pl.ds(1, CHUNK)], semc.at[slot])
        top = pltpu.make_async_copy(
            x_hbm.at[n, pl.ds(r0 - 1, 1)],
            xb.at[slot, pl.ds(0, 1)], semt.at[slot])
        bot = pltpu.make_async_copy(
            x_hbm.at[n, pl.ds(r0 + CHUNK, 1)],
            xb.at[slot, pl.ds(CHUNK + 1, 1)], semb.at[slot])
        return r0, core, top, bot

    def _issue(idx, slot):
        r0, core, top, bot = _copies(idx, slot)
        core.start()

        @pl.when(r0 > 0)
        def _():
            top.start()

        @pl.when(r0 + CHUNK < H)
        def _():
            bot.start()

    def _wait(idx, slot):
        r0, core, top, bot = _copies(idx, slot)
        core.wait()

        @pl.when(r0 > 0)
        def _():
            top.wait()

        @pl.when(r0 + CHUNK < H)
        def _():
            bot.wait()

    @pl.when(i == 0)
    def _():
        _issue(jnp.int32(0), 0)

    @pl.when(c == 0)
    def _():
        _issue(i + 1, 1)

    @pl.when(jnp.logical_and(c == 1, p < N * HW - 1))
    def _():
        _issue(i + 1, 0)

    @pl.when(c == 0)
    def _():
        _wait(i, 0)

    @pl.when(c == 1)
    def _():
        _wait(i, 1)

    @pl.when(i % (HW * NCHUNK) == 0)
    def _():
        xb[0, 0] = jnp.zeros((W, DIM), F32)

    @pl.when(i % (HW * NCHUNK) == HW * NCHUNK - 1)
    def _():
        xb[1, CHUNK + 1] = jnp.zeros((W, DIM), F32)

    xall = xb[c]
    xbf = xall.astype(BF16)
    v = jnp.dot(xbf.reshape((CHUNK + 2) * W, DIM), wv_ref[...],
                preferred_element_type=F32) + bv_ref[0]
    v3 = v.reshape(CHUNK + 2, W, DIM)
    zcol = jnp.zeros((CHUNK + 2, 1, DIM), F32)
    vp = jnp.concatenate([zcol, v3, zcol], axis=1)

    g = None
    for ky in range(3):
        for kx in range(3):
            term = vp[ky:ky + CHUNK, kx:kx + W, :] * wg_ref[ky, kx]
            g = term if g is None else g + term

    xc = xbf[1:CHUNK + 1].reshape(CHUNK * W, DIM)
    out = (jnp.dot(xc, b_ref[0], preferred_element_type=F32)
           + jnp.dot(g.reshape(CHUNK * W, DIM).astype(BF16), wo_ref[...],
                     preferred_element_type=F32)
           + cv_ref[0, 0])
    out_ref[0] = out.reshape(CHUNK, W, DIM)


def kernel(x, W_qkv, b_qkv, W_o, b_o, W_g, b_g):
    x32 = x.reshape(N, HW, ROWS, W, DIM).reshape(N * HW, ROWS, W, DIM)

    x_pool = pl.pallas_call(
        _pool_body,
        grid=(N * HW,),
        in_specs=[pl.BlockSpec((1, ROWS, W, DIM), lambda i: (i, 0, 0, 0))],
        out_specs=pl.BlockSpec((1, 16, DIM), lambda i: (i, 0, 0)),
        out_shape=jax.ShapeDtypeStruct((N * HW, 16, DIM), F32),
    )(x32)

    bqkv2 = b_qkv.reshape(1, 2 * QK_DIM + DIM)

    mm = pl.pallas_call(
        _route_body,
        in_specs=[
            pl.BlockSpec((N * HW, 16, DIM), lambda: (0, 0, 0)),
            pl.BlockSpec((DIM, 2 * QK_DIM + DIM), lambda: (0, 0)),
            pl.BlockSpec((1, 2 * QK_DIM + DIM), lambda: (0, 0)),
        ],
        out_specs=pl.BlockSpec((N * HW, QK_DIM, DIM), lambda: (0, 0, 0)),
        out_shape=jax.ShapeDtypeStruct((N * HW, QK_DIM, DIM), F32),
    )(x_pool, W_qkv, bqkv2)

    bo2 = b_o.reshape(1, DIM)
    bg2 = b_g.reshape(1, DIM)

    bmat, cvec = pl.pallas_call(
        _bmat_body,
        grid=(N * HW,),
        in_specs=[
            pl.BlockSpec((1, QK_DIM, DIM), lambda i: (i, 0, 0)),
            pl.BlockSpec((DIM, 2 * QK_DIM + DIM), lambda i: (0, 0)),
            pl.BlockSpec((1, 2 * QK_DIM + DIM), lambda i: (0, 0)),
            pl.BlockSpec((DIM, DIM), lambda i: (0, 0)),
            pl.BlockSpec((1, DIM), lambda i: (0, 0)),
            pl.BlockSpec((1, DIM), lambda i: (0, 0)),
        ],
        out_specs=[
            pl.BlockSpec((1, DIM, DIM), lambda i: (i, 0, 0)),
            pl.BlockSpec((1, 1, DIM), lambda i: (i, 0, 0)),
        ],
        out_shape=[
            jax.ShapeDtypeStruct((N * HW, DIM, DIM), F32),
            jax.ShapeDtypeStruct((N * HW, 1, DIM), F32),
        ],
    )(mm, W_qkv, bqkv2, W_o, bo2, bg2)

    wv = W_qkv[:, QK_DIM + QK_DIM:].astype(BF16)
    bv = b_qkv[QK_DIM + QK_DIM:].reshape(1, DIM)
    wg = jnp.transpose(W_g[:, 0, :, :], (1, 2, 0))
    bmat16 = bmat.astype(BF16)
    wo16 = W_o.astype(BF16)

    out = pl.pallas_call(
        _main_body,
        grid=(N * HW, NCHUNK),
        in_specs=[
            pl.BlockSpec((1, DIM, DIM), lambda p, c: (p, 0, 0)),
            pl.BlockSpec((1, 1, DIM), lambda p, c: (p, 0, 0)),
            pl.BlockSpec((DIM, DIM), lambda p, c: (0, 0)),
            pl.BlockSpec((1, DIM), lambda p, c: (0, 0)),
            pl.BlockSpec((3, 3, DIM), lambda p, c: (0, 0, 0)),
            pl.BlockSpec((DIM, DIM), lambda p, c: (0, 0)),
            pl.BlockSpec(memory_space=pl.ANY),
        ],
        out_specs=pl.BlockSpec((1, CHUNK, W, DIM),
                               lambda p, c: (p // HW, (p % HW) * NCHUNK + c,
                                             0, 0)),
        out_shape=jax.ShapeDtypeStruct((N, H, W, DIM), F32),
        scratch_shapes=[
            pltpu.VMEM((2, CHUNK + 2, W, DIM), F32),
            pltpu.SemaphoreType.DMA((2,)),
            pltpu.SemaphoreType.DMA((2,)),
            pltpu.SemaphoreType.DMA((2,)),
        ],
        compiler_params=pltpu.CompilerParams(
            dimension_semantics=("arbitrary", "arbitrary")),
    )(bmat16, cvec, wv, bv, wg, wo16, x)

    return out
